# baseline (speedup 1.0000x reference)
import functools

import jax
import jax.numpy as jnp
from jax import lax
from jax.experimental import pallas as pl
from jax.experimental.pallas import tpu as pltpu
from jax.experimental.pallas import tpu_sc as plsc

NX = 13
NODES = NX * NX
L_SIZE = 0.1
THICKNESS = 0.001
RHO = 2700.0
CP = 900.0
DT = 1.0
DX = L_SIZE / (NX - 1)
DY = L_SIZE / (NX - 1)
BOLTZ = 5.67e-08
VOL_A = RHO * CP * THICKNESS * DX * DY / DT

NW = 32
CT = 128
NCT = 2
TC_BN = 512
N0S = (0, 56, 112)
N1S = (56, 112, NODES)
H0S = (0, 40, 96)
HRS = (72, 88, NODES - 96)
TO_ROWS = 88
X_ROWS = 64


def _coef_table(K, E):
    def diag(M, k):
        return (M * jnp.eye(NODES, k=k, dtype=jnp.float32)).sum(axis=1)

    kc = diag(K, 0) + VOL_A * (-1.0)
    kl = diag(K, -1)
    kr = diag(K, 1)
    kd = diag(K, -13)
    ku = diag(K, 13)
    be = BOLTZ * diag(E, 0)
    t = jnp.stack([kc, kl, kr, kd, ku, be], axis=1)
    return jnp.repeat(t, 16, axis=1)


@functools.lru_cache(maxsize=None)
def _build_tc(n_blocks, off_blocks):
    def tc_body(to_ref, tn_ref, ht_ref, if_ref, te_ref, k_ref, cf_ref,
                out_ref):
        i = pl.program_id(0)
        to = to_ref[...]
        kto = jax.lax.dot_general(
            k_ref[...], to, (((1,), (0,)), ((), ())),
            precision=jax.lax.Precision.HIGHEST,
            preferred_element_type=jnp.float32)
        be = cf_ref[...][:, 80:81]
        to2 = to * to
        te = te_ref[...]
        te2 = te * te
        res = VOL_A * (tn_ref[...] - to) + kto
        res = res - ht_ref[...] - if_ref[...]
        res = res + be * (to2 * to2 - te2 * te2)
        part = jnp.sum(jnp.abs(res))

        @pl.when(i == 0)
        def _():
            out_ref[0, 0] = 0.0

        out_ref[0, 0] += part

    grid = (n_blocks,)
    return pl.pallas_call(
        tc_body,
        grid=grid,
        in_specs=[
            pl.BlockSpec((NODES, TC_BN), lambda i: (0, off_blocks + i)),
            pl.BlockSpec((NODES, TC_BN), lambda i: (0, off_blocks + i)),
            pl.BlockSpec((NODES, TC_BN), lambda i: (0, off_blocks + i)),
            pl.BlockSpec((NODES, TC_BN), lambda i: (0, off_blocks + i)),
            pl.BlockSpec((NODES, TC_BN), lambda i: (0, off_blocks + i)),
            pl.BlockSpec((NODES, NODES), lambda i: (0, 0)),
            pl.BlockSpec((NODES, 96), lambda i: (0, 0)),
        ],
        out_specs=pl.BlockSpec((1, 1), lambda i: (0, 0),
                               memory_space=pltpu.SMEM),
        out_shape=jax.ShapeDtypeStruct((1, 1), jnp.float32),
    )


@functools.lru_cache(maxsize=None)
def _build_sc(B):
    assert B == NW * NCT * CT
    mesh = plsc.VectorSubcoreMesh(core_axis_name="c", subcore_axis_name="s")
    units = [(ct, nh) for ct in range(NCT) for nh in range(3)]

    @functools.partial(
        pl.kernel,
        mesh=mesh,
        out_type=jax.ShapeDtypeStruct((NW, 16), jnp.float32),
        compiler_params=pltpu.CompilerParams(use_tc_tiling_on_sc=True),
        scratch_types=(
            [pltpu.VMEM((TO_ROWS, CT), jnp.float32),
             pltpu.VMEM((X_ROWS, CT), jnp.float32),
             pltpu.VMEM((X_ROWS, CT), jnp.float32),
             pltpu.VMEM((X_ROWS, CT), jnp.float32),
             pltpu.VMEM((X_ROWS, CT), jnp.float32)] * 2
            + [
                pltpu.VMEM((NODES, 96), jnp.float32),
                pltpu.VMEM((16,), jnp.float32),
                pltpu.SemaphoreType.DMA,
                pltpu.SemaphoreType.DMA,
            ]
        ),
    )
    def sc_loss(to_h, tn_h, ht_h, if_h, te_h, coef_h, out_h,
                x0, x1, x2, x3, x4, y0, y1, y2, y3, y4,
                cf, accv, semA, semB):
        wid = lax.axis_index("s") * 2 + lax.axis_index("c")
        srcs = (to_h, tn_h, ht_h, if_h, te_h)
        halves = ((x0, x1, x2, x3, x4), (y0, y1, y2, y3, y4))
        sems = (semA, semB)
        pltpu.sync_copy(coef_h, cf)
        colbase = wid * (NCT * CT)

        def copies(u):
            ct, nh = units[u]
            bufs = halves[u % 2]
            c0 = colbase + ct * CT
            out = [(srcs[0].at[pl.ds(H0S[nh], HRS[nh]), pl.ds(c0, CT)],
                    bufs[0].at[pl.ds(0, HRS[nh]), pl.ds(0, CT)])]
            nr = N1S[nh] - N0S[nh]
            for k in range(1, 5):
                out.append((srcs[k].at[pl.ds(N0S[nh], nr), pl.ds(c0, CT)],
                            bufs[k].at[pl.ds(0, nr), pl.ds(0, CT)]))
            return out

        def issue(u):
            for s, d in copies(u):
                pltpu.make_async_copy(s, d, sems[u % 2]).start()

        def drain(u):
            for s, d in copies(u):
                pltpu.make_async_copy(s, d, sems[u % 2]).wait()

        def compute(u, acc):
            ct, nh = units[u]
            gTo, gTn, gHt, gIf, gTe = halves[u % 2]
            n0, n1, h0 = N0S[nh], N1S[nh], H0S[nh]
            nr = n1 - n0
            hr = HRS[nh]
            dT = n0 - h0

            def one_node(i, a, _n0=n0, _dT=dT, _hr=hr, _nh=nh):
                kc = cf[i + _n0, pl.ds(0, 16)]
                kl = cf[i + _n0, pl.ds(16, 16)]
                kr = cf[i + _n0, pl.ds(32, 16)]
                kd = cf[i + _n0, pl.ds(48, 16)]
                ku = cf[i + _n0, pl.ds(64, 16)]
                be = cf[i + _n0, pl.ds(80, 16)]
                rT = i + _dT
                rl = jnp.maximum(rT - 1, 0) if _nh == 0 else rT - 1
                rd = jnp.maximum(rT - 13, 0) if _nh == 0 else rT - 13
                rr = jnp.minimum(rT + 1, _hr - 1) if _nh == 2 else rT + 1
                ru = jnp.minimum(rT + 13, _hr - 1) if _nh == 2 else rT + 13
                for j in range(CT // 16):
                    cb = 16 * j
                    toc = gTo[rT, pl.ds(cb, 16)]
                    tol = gTo[rl, pl.ds(cb, 16)]
                    tor = gTo[rr, pl.ds(cb, 16)]
                    tod = gTo[rd, pl.ds(cb, 16)]
                    tou = gTo[ru, pl.ds(cb, 16)]
                    tnn = gTn[i, pl.ds(cb, 16)]
                    q1 = gHt[i, pl.ds(cb, 16)]
                    q2 = gIf[i, pl.ds(cb, 16)]
                    tee = gTe[i, pl.ds(cb, 16)]
                    t = VOL_A * tnn + kc * toc
                    t = t + kl * tol + kr * tor
                    t = t + kd * tod + ku * tou
                    t = t - q1 - q2
                    to2 = toc * toc
                    te2 = tee * tee
                    t = t + be * (to2 * to2 - te2 * te2)
                    a = a + jnp.abs(t)
                return a

            def node_pair(p, a):
                a = one_node(2 * p, a)
                return one_node(2 * p + 1, a)

            acc2 = lax.fori_loop(0, nr // 2, node_pair, acc)
            if nr % 2:
                acc2 = one_node(nr - 1, acc2)
            return acc2

        acc = jnp.zeros((16,), jnp.float32)
        issue(0)
        issue(1)
        for u in range(len(units)):
            drain(u)
            if u + 2 < len(units):
                issue(u + 2)
            acc = compute(u, acc)
        accv[...] = acc
        pltpu.sync_copy(accv, out_h.at[wid])

    return sc_loss


def kernel(T_new, T_old, heaters_input, interfaces_input, Tenv, K, E):
    B = T_new.shape[0]
    coef = _coef_table(K, E)
    b_sc = NW * NCT * CT
    sc = _build_sc(b_sc)
    tT_old = T_old.T
    tT_new = T_new.T
    tHt = heaters_input.astype(jnp.float32).T
    tIf = interfaces_input.astype(jnp.float32).T
    tTe = Tenv.T
    partials = sc(tT_old, tT_new, tHt, tIf, tTe, coef)
    tc = _build_tc((B - b_sc) // TC_BN, b_sc // TC_BN)
    tc_sum = tc(tT_old, tT_new, tHt, tIf, tTe, K, coef)
    return (jnp.sum(partials) + tc_sum[0, 0]) / (B * NODES)
